# baseline (speedup 1.0000x reference)
import jax
import jax.numpy as jnp
from jax import lax
from jax.experimental import pallas as pl
from jax.experimental.pallas import tpu as pltpu
from jax.experimental.pallas import tpu_sc as plsc
from jax._src.pallas import mpmd as _plmpmd

_NUM_WORKERS = 32
_LANES = 16


def _copy_body(state_ref, w_ref, b_ref, x_ref, o_ref, arfa_ref):
    @pl.when(pl.program_id(0) == 0)
    def _():
        z = lax.dot_general(
            w_ref[...],
            state_ref[...],
            (((1,), (1,)), ((), ())),
            preferred_element_type=jnp.float32,
        )
        arfa_ref[...] = jnp.broadcast_to(
            jax.nn.sigmoid(z + b_ref[...]), arfa_ref.shape
        )

    o_ref[...] = x_ref[...]


def _make_fix_kernel(V: int, B: int, g_pad: int):
    per_w = g_pad // _NUM_WORKERS
    n_a = 128
    n_b = per_w - n_a
    mesh = plsc.VectorSubcoreMesh(core_axis_name="c", subcore_axis_name="s")

    def fix_body(x_hbm, g_hbm, arfa_hbm, skel_hbm, out_hbm,
                 idx_a, idx_b, rows_a, rows_b, arfa_v, sem, sem2):
        del skel_hbm
        c = lax.axis_index("c")
        s = lax.axis_index("s")
        wid = s * 2 + c
        base = wid * per_w

        pltpu.sync_copy(g_hbm.at[pl.ds(base, n_a)], idx_a)
        pltpu.sync_copy(g_hbm.at[pl.ds(base + n_a, n_b)], idx_b)
        pltpu.sync_copy(arfa_hbm.at[0], arfa_v)

        ga = pltpu.async_copy(x_hbm.at[idx_a], rows_a, sem)
        gb = pltpu.async_copy(x_hbm.at[idx_b], rows_b, sem2)
        ga.wait()
        gb.wait()

        chunks = B // _LANES

        def scale_a(j, carry):
            sl = pl.ds((j % chunks) * _LANES, _LANES)
            rows_a[j // chunks, sl] = rows_a[j // chunks, sl] * arfa_v[sl]
            return carry

        lax.fori_loop(0, n_a * chunks, scale_a, 0, unroll=8)

        def scale_b(j, carry):
            sl = pl.ds((j % chunks) * _LANES, _LANES)
            rows_b[j // chunks, sl] = rows_b[j // chunks, sl] * arfa_v[sl]
            return carry

        lax.fori_loop(0, n_b * chunks, scale_b, 0, unroll=8)

        sa = pltpu.async_copy(rows_a, out_hbm.at[idx_a], sem)
        sb = pltpu.async_copy(rows_b, out_hbm.at[idx_b], sem2)
        sa.wait()
        sb.wait()

    return _plmpmd._mpmd_map(
        [(mesh, fix_body)],
        jax.ShapeDtypeStruct((V, B), jnp.float32),
        input_output_aliases={3: 0},
        scratch_types=[
            pltpu.VMEM((n_a,), jnp.int32),
            pltpu.VMEM((n_b,), jnp.int32),
            pltpu.VMEM((n_a, B), jnp.float32),
            pltpu.VMEM((n_b, B), jnp.float32),
            pltpu.VMEM((B,), jnp.float32),
            pltpu.SemaphoreType.DMA,
            pltpu.SemaphoreType.DMA,
        ],
        compiler_params=pltpu.CompilerParams(needs_layout_passes=False),
    )


def kernel(output, state, grammar, W, b):
    B, _, V = output.shape
    H = state.shape[-1]
    G = grammar.shape[0]

    vblk = 14336
    n_blocks = -(-V // vblk)

    g_pad = -(-G // (_NUM_WORKERS * _LANES)) * (_NUM_WORKERS * _LANES)
    gpad = jnp.concatenate(
        [grammar, jnp.broadcast_to(grammar[:1], (g_pad - G,))]
    )

    xt = jnp.transpose(output, (1, 2, 0)).reshape(V, B)
    state2d = state.reshape(B, H)
    b2d = b.reshape(1, 1)

    skel, arfa = pl.pallas_call(
        _copy_body,
        grid=(n_blocks,),
        in_specs=[
            pl.BlockSpec((B, H), lambda i: (0, 0)),
            pl.BlockSpec((1, H), lambda i: (0, 0)),
            pl.BlockSpec((1, 1), lambda i: (0, 0)),
            pl.BlockSpec((vblk, B), lambda i: (i, 0)),
        ],
        out_specs=[
            pl.BlockSpec((vblk, B), lambda i: (i, 0)),
            pl.BlockSpec((8, 128), lambda i: (0, 0)),
        ],
        out_shape=[
            jax.ShapeDtypeStruct((V, B), jnp.float32),
            jax.ShapeDtypeStruct((8, 128), jnp.float32),
        ],
    )(state2d, W, b2d, xt)

    out_t = _make_fix_kernel(V, B, g_pad)(xt, gpad, arfa, skel)

    return jnp.transpose(out_t.reshape(1, V, B), (2, 0, 1))


# speedup vs baseline: 1.1875x; 1.1875x over previous
import functools

import jax
import jax.numpy as jnp
from jax import lax
from jax.experimental import pallas as pl
from jax.experimental.pallas import tpu as pltpu
from jax.experimental.pallas import tpu_sc as plsc

_NUM_WORKERS = 32
_LANES = 16


def _make_mask_kernel(rows: int, g_rows: int):
    rows_per_w = rows // _NUM_WORKERS
    chunk = rows_per_w * 128
    mesh = plsc.VectorSubcoreMesh(core_axis_name="c", subcore_axis_name="s")

    @functools.partial(
        pl.kernel,
        mesh=mesh,
        out_type=jax.ShapeDtypeStruct((rows, 128), jnp.float32),
        scratch_types=[
            pltpu.VMEM((g_rows, 128), jnp.int32),
            pltpu.VMEM((rows_per_w, 128), jnp.float32),
            pltpu.SemaphoreType.DMA,
        ],
        compiler_params=pltpu.CompilerParams(needs_layout_passes=False),
    )
    def mask_kernel(grammar_hbm, mask_hbm, idx_v, buf_v, sem):
        c = lax.axis_index("c")
        s = lax.axis_index("s")
        wid = s * 2 + c
        base = wid * chunk

        gcopy = pltpu.async_copy(grammar_hbm, idx_v, sem)

        zeros16 = jnp.zeros((_LANES,), jnp.float32)

        def zero_body(i, carry):
            buf_v[i // 8, pl.ds((i % 8) * _LANES, _LANES)] = zeros16
            return carry

        lax.fori_loop(0, rows_per_w * 8, zero_body, 0, unroll=8)

        gcopy.wait()

        ones16 = jnp.ones((_LANES,), jnp.float32)

        def scatter_body(j, carry):
            idx = idx_v[j // 8, pl.ds((j % 8) * _LANES, _LANES)]
            m = (idx >= base) & (idx < base + chunk)
            local = jnp.where(m, idx - base, 0)
            row = lax.shift_right_logical(local, 7)
            col = lax.bitwise_and(local, 127)
            plsc.store_scatter(buf_v, [row, col], ones16, mask=m)
            return carry

        lax.fori_loop(0, g_rows * 8, scatter_body, 0, unroll=8)

        pltpu.sync_copy(buf_v, mask_hbm.at[pl.ds(wid * rows_per_w, rows_per_w), :])

    return mask_kernel


def _blend_body(
    state_ref, w_ref, b_ref, x_ref, m_ref, o_ref, arfa_ref, mt_ref, ntiles
):
    @pl.when(pl.program_id(0) == 0)
    def _():
        z = lax.dot_general(
            w_ref[...],
            state_ref[...],
            (((1,), (1,)), ((), ())),
            preferred_element_type=jnp.float32,
        )
        arfa_ref[...] = jax.nn.sigmoid(z + b_ref[...])
        for ib in range(mt_ref.shape[0]):
            mt_ref[ib] = m_ref[ib * ntiles : (ib + 1) * ntiles, :].T

    i = pl.program_id(0)
    arfa = arfa_ref[...]
    mta = mt_ref[i]
    for t in range(ntiles):
        m_col = mta[:, t : t + 1] != 0.0
        sl = slice(t * 128, (t + 1) * 128)
        x_blk = x_ref[sl, :]
        o_ref[sl, :] = jnp.where(m_col, x_blk * arfa, x_blk)


def kernel(output, state, grammar, W, b):
    B, _, V = output.shape
    H = state.shape[-1]
    G = grammar.shape[0]

    vblk = 14336
    tiles_per_blk = vblk // 128
    n_blocks = -(-V // vblk)

    rows = -(-(n_blocks * tiles_per_blk) // (_NUM_WORKERS * 8)) * (_NUM_WORKERS * 8)
    g_rows = -(-G // 128)

    gpad = jnp.concatenate(
        [grammar, jnp.full((g_rows * 128 - G,), -1, jnp.int32)]
    ).reshape(g_rows, 128)

    mask = _make_mask_kernel(rows, g_rows)(gpad)

    xt = jnp.transpose(output, (1, 2, 0)).reshape(V, B)
    state2d = state.reshape(B, H)
    b2d = b.reshape(1, 1)

    import functools as _ft

    out_t = pl.pallas_call(
        _ft.partial(_blend_body, ntiles=tiles_per_blk),
        grid=(n_blocks,),
        in_specs=[
            pl.BlockSpec((B, H), lambda i: (0, 0)),
            pl.BlockSpec((1, H), lambda i: (0, 0)),
            pl.BlockSpec((1, 1), lambda i: (0, 0)),
            pl.BlockSpec((vblk, B), lambda i: (i, 0)),
            pl.BlockSpec((rows, 128), lambda i: (0, 0)),
        ],
        out_specs=pl.BlockSpec((vblk, B), lambda i: (i, 0)),
        out_shape=jax.ShapeDtypeStruct((V, B), jnp.float32),
        scratch_shapes=[
            pltpu.VMEM((1, B), jnp.float32),
            pltpu.VMEM((n_blocks, 128, tiles_per_blk), jnp.float32),
        ],
    )(state2d, W, b2d, xt, mask)

    return jnp.transpose(out_t.reshape(1, V, B), (2, 0, 1))
